# baseline (speedup 1.0000x reference)
import functools

import jax
import jax.numpy as jnp
from jax import lax
from jax.experimental import pallas as pl
from jax.experimental.pallas import tpu as pltpu
from jax.experimental.pallas import tpu_sc as plsc

NUM_CORES = 2
NUM_SUBCORES = 16
NW = NUM_CORES * NUM_SUBCORES

NBUF = 8


def _build_gather(batch: int, seq: int, d_model: int):
    mesh = plsc.VectorSubcoreMesh(
        core_axis_name="c", subcore_axis_name="s",
        num_cores=NUM_CORES, num_subcores=NUM_SUBCORES)
    rows_per_w = batch // NW
    n_chunks = rows_per_w
    n_groups = n_chunks // NBUF

    @functools.partial(
        pl.kernel,
        out_type=jax.ShapeDtypeStruct((batch, seq, d_model), jnp.float32),
        mesh=mesh,
        scratch_types=[
            pltpu.VMEM((n_chunks, seq), jnp.int32),
            pltpu.VMEM((NBUF, seq, d_model), jnp.float32),
            pltpu.SemaphoreType.DMA,
            pltpu.SemaphoreType.DMA,
        ],
        compiler_params=pltpu.CompilerParams(use_tc_tiling_on_sc=False),
    )
    def gather_kernel(x_hbm, table_hbm, out_hbm, idx_v, rows_v, gsem, ssem):
        wid = lax.axis_index("s") * NUM_CORES + lax.axis_index("c")
        r0 = wid * rows_per_w
        pltpu.sync_copy(x_hbm.at[pl.ds(r0, rows_per_w)], idx_v)

        def gather_desc(j, b):
            return pltpu.make_async_copy(
                table_hbm.at[idx_v.at[j]], rows_v.at[b], gsem)

        def store_desc(j, b):
            return pltpu.make_async_copy(
                rows_v.at[b], out_hbm.at[r0 + j], ssem)

        for b in range(NBUF):
            gather_desc(b, b).start()

        @pl.loop(0, n_groups)
        def _(g):
            j0 = g * NBUF
            for b in range(NBUF):
                gather_desc(j0 + b, b).wait()
                store_desc(j0 + b, b).start()
            @pl.when(g + 1 < n_groups)
            def _():
                for b in range(NBUF):
                    store_desc(j0 + b, b).wait()
                    gather_desc(j0 + NBUF + b, b).start()

            @pl.when(g + 1 == n_groups)
            def _():
                for b in range(NBUF):
                    store_desc(j0 + b, b).wait()

    return gather_kernel


def kernel(x, embedding_matrix):
    batch, seq = x.shape
    _, d_model = embedding_matrix.shape
    gather = _build_gather(batch, seq, d_model)
    return gather(x, embedding_matrix)
